# Initial kernel scaffold; baseline (speedup 1.0000x reference)
import jax
import jax.numpy as jnp
from jax.experimental import pallas as pl


def kernel(inputs, adj, W1, a1, Wt, W2, a2):
    raise NotImplementedError("write your pallas kernel here")


# speedup vs baseline: 3.1706x; 3.1706x over previous
import functools

import jax
import jax.numpy as jnp
from jax.experimental import pallas as pl
from jax.experimental.pallas import tpu as pltpu

ALPHA_SLOPE = 0.2
MASKED = -9e15

BI1, BJ1 = 256, 1024
BI2, BJ2 = 256, 1024
BI0 = 512


def _leaky(x):
    return jnp.maximum(x, ALPHA_SLOPE * x)


def _proj_kernel(n_rows, x_ref, w1_ref, a1_ref, wh_ref, s1a_ref, s1b_ref):
    i = pl.program_id(0)
    f = w1_ref.shape[1]
    bi = x_ref.shape[0]
    rows = i * bi + jax.lax.broadcasted_iota(jnp.int32, (bi, 1), 0)
    wh = jnp.dot(x_ref[...], w1_ref[...], preferred_element_type=jnp.float32)
    wh = jnp.where(rows < n_rows, wh, 0.0)
    wh_ref[...] = wh
    s1a_ref[...] = jnp.dot(wh, a1_ref[0:f, :], preferred_element_type=jnp.float32)
    s1b = jax.lax.dot_general(a1_ref[f:, :], wh, (((0,), (1,)), ((), ())),
                              preferred_element_type=jnp.float32)
    cols = i * bi + jax.lax.broadcasted_iota(jnp.int32, (1, bi), 1)
    s1b_ref[...] = jnp.where(cols < n_rows, s1b, -jnp.inf)


def _l1_kernel(n_rows, adj_ref, wh_ref, s1a_ref, s1b_ref, neg_ref, colv_ref,
               wt_ref, w2_ref, a2_ref,
               res_ref, xwt_ref, wh2_ref, s2a_ref, s2b_ref, maskp_ref,
               acc_ref, m_ref, l_ref):
    i = pl.program_id(0)
    j = pl.program_id(1)
    nj = pl.num_programs(1)
    bi, bj = adj_ref.shape
    c = w2_ref.shape[1]

    @pl.when(j == 0)
    def _():
        acc_ref[...] = jnp.zeros_like(acc_ref)
        m_ref[...] = jnp.full_like(m_ref, MASKED)
        l_ref[...] = jnp.zeros_like(l_ref)

    adj = adj_ref[...]
    mask = jnp.logical_and(adj > 0, colv_ref[...] > 0)
    maskp_ref[...] = mask.astype(jnp.int8)
    e = _leaky(s1a_ref[...] + s1b_ref[...])
    score = jnp.where(mask, e, neg_ref[...])
    m_old = m_ref[...]
    m_new = jnp.maximum(m_old, jnp.max(score, axis=1, keepdims=True))
    p = jnp.exp(score - m_new)
    corr = jnp.exp(m_old - m_new)
    l_ref[...] = l_ref[...] * corr + jnp.sum(p, axis=1, keepdims=True)
    whj = wh_ref[pl.ds(j * bj, bj), :]
    acc_ref[...] = acc_ref[...] * corr + jnp.dot(
        p, whj, preferred_element_type=jnp.float32)
    m_ref[...] = m_new

    @pl.when(j == nj - 1)
    def _():
        rows = i * bi + jax.lax.broadcasted_iota(jnp.int32, (bi, 1), 0)
        h = acc_ref[...] / l_ref[...]
        h = jnp.where(h > 0, h, jnp.exp(h) - 1.0)
        h = jnp.where(rows < n_rows, h, 0.0)
        res_ref[...] = h
        xwt_ref[...] = jnp.dot(h, wt_ref[...], preferred_element_type=jnp.float32)
        wh2 = jnp.dot(h, w2_ref[...], preferred_element_type=jnp.float32)
        wh2_ref[...] = wh2
        s2a_ref[...] = jnp.dot(wh2, a2_ref[0:c, :], preferred_element_type=jnp.float32)
        s2b = jax.lax.dot_general(a2_ref[c:, :], wh2, (((0,), (1,)), ((), ())),
                                  preferred_element_type=jnp.float32)
        cols = i * bi + jax.lax.broadcasted_iota(jnp.int32, (1, bi), 1)
        s2b_ref[...] = jnp.where(cols < n_rows, s2b, -jnp.inf)


def _l2_kernel(maskp_ref, res_ref, xwt_ref, wh2_ref, s2a_ref, s2b_ref, neg_ref,
               out_ref, acc_ref, mt_ref, lt_ref, me_ref, le_ref):
    j = pl.program_id(1)
    nj = pl.num_programs(1)
    bj = maskp_ref.shape[1]

    @pl.when(j == 0)
    def _():
        acc_ref[...] = jnp.zeros_like(acc_ref)
        mt_ref[...] = jnp.full_like(mt_ref, MASKED)
        lt_ref[...] = jnp.zeros_like(lt_ref)
        me_ref[...] = jnp.full_like(me_ref, MASKED)
        le_ref[...] = jnp.zeros_like(le_ref)

    mask = maskp_ref[...].astype(jnp.int32) > 0
    neg = neg_ref[...]
    xj = res_ref[pl.ds(j * bj, bj), :]
    t = jax.lax.dot_general(xwt_ref[...], xj, (((1,), (1,)), ((), ())),
                            preferred_element_type=jnp.float32)
    st = jnp.where(mask, _leaky(t), neg)
    se = jnp.where(mask, _leaky(s2a_ref[...] + s2b_ref[...]), neg)
    mt_old = mt_ref[...]
    me_old = me_ref[...]
    mt_new = jnp.maximum(mt_old, jnp.max(st, axis=1, keepdims=True))
    me_new = jnp.maximum(me_old, jnp.max(se, axis=1, keepdims=True))
    pt = jnp.exp(st - mt_new)
    pe = jnp.exp(se - me_new)
    ct = jnp.exp(mt_old - mt_new)
    ce = jnp.exp(me_old - me_new)
    lt_ref[...] = lt_ref[...] * ct + jnp.sum(pt, axis=1, keepdims=True)
    le_ref[...] = le_ref[...] * ce + jnp.sum(pe, axis=1, keepdims=True)
    wh2j = wh2_ref[pl.ds(j * bj, bj), :]
    acc_ref[...] = acc_ref[...] * (ct * ce) + jnp.dot(
        pt * pe, wh2j, preferred_element_type=jnp.float32)
    mt_ref[...] = mt_new
    me_ref[...] = me_new

    @pl.when(j == nj - 1)
    def _():
        h = acc_ref[...] / (lt_ref[...] * le_ref[...])
        mx = jnp.max(h, axis=1, keepdims=True)
        lse = mx + jnp.log(jnp.sum(jnp.exp(h - mx), axis=1, keepdims=True))
        out_ref[...] = h - lse


@jax.jit
def kernel(inputs, adj, W1, a1, Wt, W2, a2):
    n, feat = inputs.shape
    h2 = W1.shape[1]
    c = W2.shape[1]
    npad = pl.cdiv(n, BJ1) * BJ1

    ids = jax.lax.iota(jnp.int32, npad)[None, :]
    colv = (ids < n).astype(jnp.float32)
    neg = jnp.where(ids < n, jnp.float32(MASKED), -jnp.inf)

    wh1, s1a, s1b = pl.pallas_call(
        functools.partial(_proj_kernel, n),
        grid=(npad // BI0,),
        in_specs=[
            pl.BlockSpec((BI0, feat), lambda i: (i, 0)),
            pl.BlockSpec((feat, h2), lambda i: (0, 0)),
            pl.BlockSpec((2 * h2, 1), lambda i: (0, 0)),
        ],
        out_specs=[
            pl.BlockSpec((BI0, h2), lambda i: (i, 0)),
            pl.BlockSpec((BI0, 1), lambda i: (i, 0)),
            pl.BlockSpec((1, BI0), lambda i: (0, i)),
        ],
        out_shape=[
            jax.ShapeDtypeStruct((npad, h2), jnp.float32),
            jax.ShapeDtypeStruct((npad, 1), jnp.float32),
            jax.ShapeDtypeStruct((1, npad), jnp.float32),
        ],
    )(inputs, W1, a1)

    res, xwt, wh2, s2a, s2b, maskp = pl.pallas_call(
        functools.partial(_l1_kernel, n),
        grid=(npad // BI1, npad // BJ1),
        in_specs=[
            pl.BlockSpec((BI1, BJ1), lambda i, j: (i, j)),
            pl.BlockSpec((npad, h2), lambda i, j: (0, 0)),
            pl.BlockSpec((BI1, 1), lambda i, j: (i, 0)),
            pl.BlockSpec((1, BJ1), lambda i, j: (0, j)),
            pl.BlockSpec((1, BJ1), lambda i, j: (0, j)),
            pl.BlockSpec((1, BJ1), lambda i, j: (0, j)),
            pl.BlockSpec((h2, h2), lambda i, j: (0, 0)),
            pl.BlockSpec((h2, c), lambda i, j: (0, 0)),
            pl.BlockSpec((2 * c, 1), lambda i, j: (0, 0)),
        ],
        out_specs=[
            pl.BlockSpec((BI1, h2), lambda i, j: (i, 0)),
            pl.BlockSpec((BI1, h2), lambda i, j: (i, 0)),
            pl.BlockSpec((BI1, c), lambda i, j: (i, 0)),
            pl.BlockSpec((BI1, 1), lambda i, j: (i, 0)),
            pl.BlockSpec((1, BI1), lambda i, j: (0, i)),
            pl.BlockSpec((BI1, BJ1), lambda i, j: (i, j)),
        ],
        out_shape=[
            jax.ShapeDtypeStruct((npad, h2), jnp.float32),
            jax.ShapeDtypeStruct((npad, h2), jnp.float32),
            jax.ShapeDtypeStruct((npad, c), jnp.float32),
            jax.ShapeDtypeStruct((npad, 1), jnp.float32),
            jax.ShapeDtypeStruct((1, npad), jnp.float32),
            jax.ShapeDtypeStruct((npad, npad), jnp.int8),
        ],
        scratch_shapes=[
            pltpu.VMEM((BI1, h2), jnp.float32),
            pltpu.VMEM((BI1, 1), jnp.float32),
            pltpu.VMEM((BI1, 1), jnp.float32),
        ],
    )(adj, wh1, s1a, s1b, neg, colv, Wt, W2, a2)

    out = pl.pallas_call(
        _l2_kernel,
        grid=(npad // BI2, npad // BJ2),
        in_specs=[
            pl.BlockSpec((BI2, BJ2), lambda i, j: (i, j)),
            pl.BlockSpec((npad, h2), lambda i, j: (0, 0)),
            pl.BlockSpec((BI2, h2), lambda i, j: (i, 0)),
            pl.BlockSpec((npad, c), lambda i, j: (0, 0)),
            pl.BlockSpec((BI2, 1), lambda i, j: (i, 0)),
            pl.BlockSpec((1, BJ2), lambda i, j: (0, j)),
            pl.BlockSpec((1, BJ2), lambda i, j: (0, j)),
        ],
        out_specs=pl.BlockSpec((BI2, c), lambda i, j: (i, 0)),
        out_shape=jax.ShapeDtypeStruct((npad, c), jnp.float32),
        scratch_shapes=[
            pltpu.VMEM((BI2, c), jnp.float32),
            pltpu.VMEM((BI2, 1), jnp.float32),
            pltpu.VMEM((BI2, 1), jnp.float32),
            pltpu.VMEM((BI2, 1), jnp.float32),
            pltpu.VMEM((BI2, 1), jnp.float32),
        ],
    )(maskp, res, xwt, wh2, s2a, s2b, neg)

    return out[:n]


# speedup vs baseline: 3.2879x; 1.0370x over previous
import functools

import jax
import jax.numpy as jnp
from jax.experimental import pallas as pl
from jax.experimental.pallas import tpu as pltpu

ALPHA_SLOPE = 0.2
MASKED = -9e15

BI1, BJ1 = 256, 1024
BI2, BJ2 = 256, 1024
BI0 = 512


def _leaky(x):
    return jnp.maximum(x, ALPHA_SLOPE * x)


def _proj_kernel(n_rows, x_ref, w1_ref, a1_ref, wh_ref, s1a_ref, s1b_ref):
    i = pl.program_id(0)
    f = w1_ref.shape[1]
    bi = x_ref.shape[0]
    rows = i * bi + jax.lax.broadcasted_iota(jnp.int32, (bi, 1), 0)
    wh = jnp.dot(x_ref[...], w1_ref[...], preferred_element_type=jnp.float32)
    wh = jnp.where(rows < n_rows, wh, 0.0)
    wh_ref[...] = wh.astype(jnp.bfloat16)
    s1a_ref[...] = jnp.dot(wh, a1_ref[0:f, :], preferred_element_type=jnp.float32)
    s1b = jax.lax.dot_general(a1_ref[f:, :], wh, (((0,), (1,)), ((), ())),
                              preferred_element_type=jnp.float32)
    cols = i * bi + jax.lax.broadcasted_iota(jnp.int32, (1, bi), 1)
    s1b_ref[...] = jnp.where(cols < n_rows, s1b, -jnp.inf)


def _l1_kernel(n_rows, adj_ref, wh_ref, s1a_ref, s1b_ref, neg_ref,
               wt_ref, w2_ref, a2_ref,
               res_ref, xwt_ref, wh2_ref, s2a_ref, s2b_ref,
               acc_ref, m_ref, l_ref):
    i = pl.program_id(0)
    j = pl.program_id(1)
    nj = pl.num_programs(1)
    bi, bj = adj_ref.shape
    c = w2_ref.shape[1]

    @pl.when(j == 0)
    def _():
        acc_ref[...] = jnp.zeros_like(acc_ref)
        m_ref[...] = jnp.full_like(m_ref, MASKED)
        l_ref[...] = jnp.zeros_like(l_ref)

    e = _leaky(s1a_ref[...] + s1b_ref[...])
    score = jnp.where(adj_ref[...] > 0, e, neg_ref[...])
    m_old = m_ref[...]
    m_new = jnp.maximum(m_old, jnp.max(score, axis=1, keepdims=True))
    p = jnp.exp(score - m_new)
    corr = jnp.exp(m_old - m_new)
    l_ref[...] = l_ref[...] * corr + jnp.sum(p, axis=1, keepdims=True)
    whj = wh_ref[pl.ds(j * bj, bj), :]
    acc_ref[...] = acc_ref[...] * corr + jnp.dot(
        p.astype(jnp.bfloat16), whj, preferred_element_type=jnp.float32)
    m_ref[...] = m_new

    @pl.when(j == nj - 1)
    def _():
        rows = i * bi + jax.lax.broadcasted_iota(jnp.int32, (bi, 1), 0)
        h = acc_ref[...] / l_ref[...]
        h = jnp.where(h > 0, h, jnp.exp(h) - 1.0)
        h = jnp.where(rows < n_rows, h, 0.0)
        res_ref[...] = h
        xwt_ref[...] = jnp.dot(h, wt_ref[...], preferred_element_type=jnp.float32)
        wh2 = jnp.dot(h, w2_ref[...], preferred_element_type=jnp.float32)
        wh2_ref[...] = wh2.astype(jnp.bfloat16)
        s2a_ref[...] = jnp.dot(wh2, a2_ref[0:c, :], preferred_element_type=jnp.float32)
        s2b = jax.lax.dot_general(a2_ref[c:, :], wh2, (((0,), (1,)), ((), ())),
                                  preferred_element_type=jnp.float32)
        cols = i * bi + jax.lax.broadcasted_iota(jnp.int32, (1, bi), 1)
        s2b_ref[...] = jnp.where(cols < n_rows, s2b, -jnp.inf)


def _l2_kernel(adj_ref, res_ref, xwt_ref, wh2_ref, s2a_ref, s2b_ref, neg_ref,
               zcol_ref, out_ref, acc_ref, mt_ref, lt_ref, me_ref, le_ref):
    j = pl.program_id(1)
    nj = pl.num_programs(1)
    bj = adj_ref.shape[1]

    @pl.when(j == 0)
    def _():
        acc_ref[...] = jnp.zeros_like(acc_ref)
        mt_ref[...] = jnp.full_like(mt_ref, MASKED)
        lt_ref[...] = jnp.zeros_like(lt_ref)
        me_ref[...] = jnp.full_like(me_ref, MASKED)
        le_ref[...] = jnp.zeros_like(le_ref)

    mask = adj_ref[...] > 0
    neg = neg_ref[...]
    xj = res_ref[pl.ds(j * bj, bj), :]
    t = jax.lax.dot_general(xwt_ref[...], xj, (((1,), (1,)), ((), ())),
                            preferred_element_type=jnp.float32)
    st = jnp.where(mask, _leaky(t) + zcol_ref[...], neg)
    se = jnp.where(mask, _leaky(s2a_ref[...] + s2b_ref[...]), neg)
    mt_old = mt_ref[...]
    me_old = me_ref[...]
    mt_new = jnp.maximum(mt_old, jnp.max(st, axis=1, keepdims=True))
    me_new = jnp.maximum(me_old, jnp.max(se, axis=1, keepdims=True))
    pt = jnp.exp(st - mt_new)
    pe = jnp.exp(se - me_new)
    ct = jnp.exp(mt_old - mt_new)
    ce = jnp.exp(me_old - me_new)
    lt_ref[...] = lt_ref[...] * ct + jnp.sum(pt, axis=1, keepdims=True)
    le_ref[...] = le_ref[...] * ce + jnp.sum(pe, axis=1, keepdims=True)
    wh2j = wh2_ref[pl.ds(j * bj, bj), :]
    acc_ref[...] = acc_ref[...] * (ct * ce) + jnp.dot(
        (pt * pe).astype(jnp.bfloat16), wh2j, preferred_element_type=jnp.float32)
    mt_ref[...] = mt_new
    me_ref[...] = me_new

    @pl.when(j == nj - 1)
    def _():
        h = acc_ref[...] / (lt_ref[...] * le_ref[...])
        mx = jnp.max(h, axis=1, keepdims=True)
        lse = mx + jnp.log(jnp.sum(jnp.exp(h - mx), axis=1, keepdims=True))
        out_ref[...] = h - lse


@jax.jit
def kernel(inputs, adj, W1, a1, Wt, W2, a2):
    n, feat = inputs.shape
    h2 = W1.shape[1]
    c = W2.shape[1]
    npad = pl.cdiv(n, BJ1) * BJ1

    ids = jax.lax.iota(jnp.int32, npad)[None, :]
    neg = jnp.where(ids < n, jnp.float32(MASKED), -jnp.inf)
    zcol = jnp.where(ids < n, jnp.float32(0.0), -jnp.inf)

    wh1, s1a, s1b = pl.pallas_call(
        functools.partial(_proj_kernel, n),
        grid=(npad // BI0,),
        in_specs=[
            pl.BlockSpec((BI0, feat), lambda i: (i, 0)),
            pl.BlockSpec((feat, h2), lambda i: (0, 0)),
            pl.BlockSpec((2 * h2, 1), lambda i: (0, 0)),
        ],
        out_specs=[
            pl.BlockSpec((BI0, h2), lambda i: (i, 0)),
            pl.BlockSpec((BI0, 1), lambda i: (i, 0)),
            pl.BlockSpec((1, BI0), lambda i: (0, i)),
        ],
        out_shape=[
            jax.ShapeDtypeStruct((npad, h2), jnp.bfloat16),
            jax.ShapeDtypeStruct((npad, 1), jnp.float32),
            jax.ShapeDtypeStruct((1, npad), jnp.float32),
        ],
    )(inputs, W1, a1)

    res, xwt, wh2, s2a, s2b = pl.pallas_call(
        functools.partial(_l1_kernel, n),
        grid=(npad // BI1, npad // BJ1),
        in_specs=[
            pl.BlockSpec((BI1, BJ1), lambda i, j: (i, j)),
            pl.BlockSpec((npad, h2), lambda i, j: (0, 0)),
            pl.BlockSpec((BI1, 1), lambda i, j: (i, 0)),
            pl.BlockSpec((1, BJ1), lambda i, j: (0, j)),
            pl.BlockSpec((1, BJ1), lambda i, j: (0, j)),
            pl.BlockSpec((h2, h2), lambda i, j: (0, 0)),
            pl.BlockSpec((h2, c), lambda i, j: (0, 0)),
            pl.BlockSpec((2 * c, 1), lambda i, j: (0, 0)),
        ],
        out_specs=[
            pl.BlockSpec((BI1, h2), lambda i, j: (i, 0)),
            pl.BlockSpec((BI1, h2), lambda i, j: (i, 0)),
            pl.BlockSpec((BI1, c), lambda i, j: (i, 0)),
            pl.BlockSpec((BI1, 1), lambda i, j: (i, 0)),
            pl.BlockSpec((1, BI1), lambda i, j: (0, i)),
        ],
        out_shape=[
            jax.ShapeDtypeStruct((npad, h2), jnp.float32),
            jax.ShapeDtypeStruct((npad, h2), jnp.float32),
            jax.ShapeDtypeStruct((npad, c), jnp.bfloat16),
            jax.ShapeDtypeStruct((npad, 1), jnp.float32),
            jax.ShapeDtypeStruct((1, npad), jnp.float32),
        ],
        scratch_shapes=[
            pltpu.VMEM((BI1, h2), jnp.float32),
            pltpu.VMEM((BI1, 1), jnp.float32),
            pltpu.VMEM((BI1, 1), jnp.float32),
        ],
    )(adj, wh1, s1a, s1b, neg, Wt, W2, a2)

    out = pl.pallas_call(
        _l2_kernel,
        grid=(npad // BI2, npad // BJ2),
        in_specs=[
            pl.BlockSpec((BI2, BJ2), lambda i, j: (i, j)),
            pl.BlockSpec((npad, h2), lambda i, j: (0, 0)),
            pl.BlockSpec((BI2, h2), lambda i, j: (i, 0)),
            pl.BlockSpec((npad, c), lambda i, j: (0, 0)),
            pl.BlockSpec((BI2, 1), lambda i, j: (i, 0)),
            pl.BlockSpec((1, BJ2), lambda i, j: (0, j)),
            pl.BlockSpec((1, BJ2), lambda i, j: (0, j)),
            pl.BlockSpec((1, BJ2), lambda i, j: (0, j)),
        ],
        out_specs=pl.BlockSpec((BI2, c), lambda i, j: (i, 0)),
        out_shape=jax.ShapeDtypeStruct((npad, c), jnp.float32),
        scratch_shapes=[
            pltpu.VMEM((BI2, c), jnp.float32),
            pltpu.VMEM((BI2, 1), jnp.float32),
            pltpu.VMEM((BI2, 1), jnp.float32),
            pltpu.VMEM((BI2, 1), jnp.float32),
            pltpu.VMEM((BI2, 1), jnp.float32),
        ],
    )(adj, res, xwt, wh2, s2a, s2b, neg, zcol)

    return out[:n]


# speedup vs baseline: 3.5662x; 1.0847x over previous
import functools

import jax
import jax.numpy as jnp
from jax.experimental import pallas as pl
from jax.experimental.pallas import tpu as pltpu

ALPHA_SLOPE = 0.2
NEG_INF = float("-inf")

BI1, BJ1 = 256, 1024
BI2, BJ2 = 256, 1024
BI0 = 512


def _leaky(x):
    return jnp.maximum(x, ALPHA_SLOPE * x)


def _proj_kernel(n_rows, x_ref, w1_ref, a1_ref,
                 wh_ref, s1a_ref, s1b_ref, maxb_ref, whsum_ref,
                 whsum_s, maxb_s):
    i = pl.program_id(0)
    ni = pl.num_programs(0)
    f = w1_ref.shape[1]
    bi = x_ref.shape[0]
    rows = i * bi + jax.lax.broadcasted_iota(jnp.int32, (bi, 1), 0)
    wh = jnp.dot(x_ref[...], w1_ref[...], preferred_element_type=jnp.float32)
    wh = jnp.where(rows < n_rows, wh, 0.0)
    wh_ref[...] = wh.astype(jnp.bfloat16)
    s1a_ref[...] = jnp.dot(wh, a1_ref[0:f, :], preferred_element_type=jnp.float32)
    s1b = jax.lax.dot_general(a1_ref[f:, :], wh, (((0,), (1,)), ((), ())),
                              preferred_element_type=jnp.float32)
    cols = i * bi + jax.lax.broadcasted_iota(jnp.int32, (1, bi), 1)
    s1b = jnp.where(cols < n_rows, s1b, NEG_INF)
    s1b_ref[...] = s1b

    @pl.when(i == 0)
    def _():
        whsum_s[...] = jnp.zeros_like(whsum_s)
        maxb_s[...] = jnp.full_like(maxb_s, NEG_INF)

    whsum_s[...] += jnp.sum(wh, axis=0, keepdims=True)
    maxb_s[...] = jnp.maximum(maxb_s[...], jnp.max(s1b, axis=1, keepdims=True))

    @pl.when(i == ni - 1)
    def _():
        whsum_ref[...] = whsum_s[...]
        maxb_ref[...] = maxb_s[...]


def _l1_kernel(n_rows, adj_ref, wh_ref, s1a_ref, s1b_ref, maxb_ref, whsum_ref,
               wt_ref, w2_ref, a2_ref,
               res_ref, xwt_ref, wh2_ref, s2a_ref, s2b_ref, normx_ref,
               gnorm_ref, maxs2b_ref, wh2sum_ref,
               acc_ref, l_ref, wh2s_s, gnorm_s, maxs2b_s):
    i = pl.program_id(0)
    j = pl.program_id(1)
    ni = pl.num_programs(0)
    nj = pl.num_programs(1)
    bi, bj = adj_ref.shape
    c = w2_ref.shape[1]

    @pl.when(j == 0)
    def _():
        acc_ref[...] = jnp.zeros_like(acc_ref)
        l_ref[...] = jnp.zeros_like(l_ref)

    m0 = _leaky(s1a_ref[...] + maxb_ref[...])
    e = _leaky(s1a_ref[...] + s1b_ref[...])
    p = jnp.exp(jnp.where(adj_ref[...] > 0, e, NEG_INF) - m0)
    l_ref[...] += jnp.sum(p, axis=1, keepdims=True)
    whj = wh_ref[pl.ds(j * bj, bj), :]
    acc_ref[...] += jnp.dot(p.astype(jnp.bfloat16), whj,
                            preferred_element_type=jnp.float32)

    @pl.when(j == nj - 1)
    def _():
        rows = i * bi + jax.lax.broadcasted_iota(jnp.int32, (bi, 1), 0)
        l = l_ref[...]
        h = jnp.where(l > 0, acc_ref[...] / l, whsum_ref[...] / n_rows)
        h = jnp.where(h > 0, h, jnp.exp(h) - 1.0)
        h = jnp.where(rows < n_rows, h, 0.0)
        res_ref[...] = h
        xwt = jnp.dot(h, wt_ref[...], preferred_element_type=jnp.float32)
        xwt_ref[...] = xwt
        normx_ref[...] = jnp.sqrt(jnp.sum(xwt * xwt, axis=1, keepdims=True))
        wh2 = jnp.dot(h, w2_ref[...], preferred_element_type=jnp.float32)
        wh2_ref[...] = wh2.astype(jnp.bfloat16)
        s2a_ref[...] = jnp.dot(wh2, a2_ref[0:c, :], preferred_element_type=jnp.float32)
        s2b = jax.lax.dot_general(a2_ref[c:, :], wh2, (((0,), (1,)), ((), ())),
                                  preferred_element_type=jnp.float32)
        cols = i * bi + jax.lax.broadcasted_iota(jnp.int32, (1, bi), 1)
        s2b = jnp.where(cols < n_rows, s2b, NEG_INF)
        s2b_ref[...] = s2b

        @pl.when(i == 0)
        def _():
            wh2s_s[...] = jnp.zeros_like(wh2s_s)
            gnorm_s[...] = jnp.zeros_like(gnorm_s)
            maxs2b_s[...] = jnp.full_like(maxs2b_s, NEG_INF)

        wh2s_s[...] += jnp.sum(wh2, axis=0, keepdims=True)
        rn = jnp.sqrt(jnp.sum(h * h, axis=1, keepdims=True))
        gnorm_s[...] = jnp.maximum(
            gnorm_s[...], jnp.max(rn, axis=0, keepdims=True))
        maxs2b_s[...] = jnp.maximum(
            maxs2b_s[...], jnp.max(s2b, axis=1, keepdims=True))

        @pl.when(i == ni - 1)
        def _():
            wh2sum_ref[...] = wh2s_s[...]
            gnorm_ref[...] = gnorm_s[...]
            maxs2b_ref[...] = maxs2b_s[...]


def _l2_kernel(n_rows, adj_ref, res_ref, xwt_ref, wh2_ref, s2a_ref, s2b_ref,
               normx_ref, gnorm_ref, maxs2b_ref, wh2sum_ref, zcol_ref,
               out_ref, acc_ref, lt_ref, le_ref):
    j = pl.program_id(1)
    nj = pl.num_programs(1)
    bj = adj_ref.shape[1]

    @pl.when(j == 0)
    def _():
        acc_ref[...] = jnp.zeros_like(acc_ref)
        lt_ref[...] = jnp.zeros_like(lt_ref)
        le_ref[...] = jnp.zeros_like(le_ref)

    mt = _leaky(normx_ref[...] * gnorm_ref[...])
    me = _leaky(s2a_ref[...] + maxs2b_ref[...])
    mask = adj_ref[...] > 0
    xj = res_ref[pl.ds(j * bj, bj), :]
    t = jax.lax.dot_general(xwt_ref[...], xj, (((1,), (1,)), ((), ())),
                            preferred_element_type=jnp.float32)
    pt = jnp.exp(jnp.where(mask, _leaky(t) + zcol_ref[...], NEG_INF) - mt)
    e2 = _leaky(s2a_ref[...] + s2b_ref[...])
    pe = jnp.exp(jnp.where(mask, e2, NEG_INF) - me)
    lt_ref[...] += jnp.sum(pt, axis=1, keepdims=True)
    le_ref[...] += jnp.sum(pe, axis=1, keepdims=True)
    wh2j = wh2_ref[pl.ds(j * bj, bj), :]
    acc_ref[...] += jnp.dot((pt * pe).astype(jnp.bfloat16), wh2j,
                            preferred_element_type=jnp.float32)

    @pl.when(j == nj - 1)
    def _():
        denom = lt_ref[...] * le_ref[...]
        uni = wh2sum_ref[...] / jnp.float32(n_rows * n_rows)
        h = jnp.where(denom > 0, acc_ref[...] / denom, uni)
        mx = jnp.max(h, axis=1, keepdims=True)
        lse = mx + jnp.log(jnp.sum(jnp.exp(h - mx), axis=1, keepdims=True))
        out_ref[...] = h - lse


@jax.jit
def kernel(inputs, adj, W1, a1, Wt, W2, a2):
    n, feat = inputs.shape
    h2 = W1.shape[1]
    c = W2.shape[1]
    npad = pl.cdiv(n, BJ1) * BJ1

    ids = jax.lax.iota(jnp.int32, npad)[None, :]
    zcol = jnp.where(ids < n, jnp.float32(0.0), NEG_INF)

    wh1, s1a, s1b, maxb, whsum = pl.pallas_call(
        functools.partial(_proj_kernel, n),
        grid=(npad // BI0,),
        in_specs=[
            pl.BlockSpec((BI0, feat), lambda i: (i, 0)),
            pl.BlockSpec((feat, h2), lambda i: (0, 0)),
            pl.BlockSpec((2 * h2, 1), lambda i: (0, 0)),
        ],
        out_specs=[
            pl.BlockSpec((BI0, h2), lambda i: (i, 0)),
            pl.BlockSpec((BI0, 1), lambda i: (i, 0)),
            pl.BlockSpec((1, BI0), lambda i: (0, i)),
            pl.BlockSpec((1, 1), lambda i: (0, 0)),
            pl.BlockSpec((1, h2), lambda i: (0, 0)),
        ],
        out_shape=[
            jax.ShapeDtypeStruct((npad, h2), jnp.bfloat16),
            jax.ShapeDtypeStruct((npad, 1), jnp.float32),
            jax.ShapeDtypeStruct((1, npad), jnp.float32),
            jax.ShapeDtypeStruct((1, 1), jnp.float32),
            jax.ShapeDtypeStruct((1, h2), jnp.float32),
        ],
        scratch_shapes=[
            pltpu.VMEM((1, h2), jnp.float32),
            pltpu.VMEM((1, 1), jnp.float32),
        ],
    )(inputs, W1, a1)

    (res, xwt, wh2, s2a, s2b, normx, gnorm, maxs2b, wh2sum) = pl.pallas_call(
        functools.partial(_l1_kernel, n),
        grid=(npad // BI1, npad // BJ1),
        in_specs=[
            pl.BlockSpec((BI1, BJ1), lambda i, j: (i, j)),
            pl.BlockSpec((npad, h2), lambda i, j: (0, 0)),
            pl.BlockSpec((BI1, 1), lambda i, j: (i, 0)),
            pl.BlockSpec((1, BJ1), lambda i, j: (0, j)),
            pl.BlockSpec((1, 1), lambda i, j: (0, 0)),
            pl.BlockSpec((1, h2), lambda i, j: (0, 0)),
            pl.BlockSpec((h2, h2), lambda i, j: (0, 0)),
            pl.BlockSpec((h2, c), lambda i, j: (0, 0)),
            pl.BlockSpec((2 * c, 1), lambda i, j: (0, 0)),
        ],
        out_specs=[
            pl.BlockSpec((BI1, h2), lambda i, j: (i, 0)),
            pl.BlockSpec((BI1, h2), lambda i, j: (i, 0)),
            pl.BlockSpec((BI1, c), lambda i, j: (i, 0)),
            pl.BlockSpec((BI1, 1), lambda i, j: (i, 0)),
            pl.BlockSpec((1, BI1), lambda i, j: (0, i)),
            pl.BlockSpec((BI1, 1), lambda i, j: (i, 0)),
            pl.BlockSpec((1, 1), lambda i, j: (0, 0)),
            pl.BlockSpec((1, 1), lambda i, j: (0, 0)),
            pl.BlockSpec((1, c), lambda i, j: (0, 0)),
        ],
        out_shape=[
            jax.ShapeDtypeStruct((npad, h2), jnp.float32),
            jax.ShapeDtypeStruct((npad, h2), jnp.float32),
            jax.ShapeDtypeStruct((npad, c), jnp.bfloat16),
            jax.ShapeDtypeStruct((npad, 1), jnp.float32),
            jax.ShapeDtypeStruct((1, npad), jnp.float32),
            jax.ShapeDtypeStruct((npad, 1), jnp.float32),
            jax.ShapeDtypeStruct((1, 1), jnp.float32),
            jax.ShapeDtypeStruct((1, 1), jnp.float32),
            jax.ShapeDtypeStruct((1, c), jnp.float32),
        ],
        scratch_shapes=[
            pltpu.VMEM((BI1, h2), jnp.float32),
            pltpu.VMEM((BI1, 1), jnp.float32),
            pltpu.VMEM((1, c), jnp.float32),
            pltpu.VMEM((1, 1), jnp.float32),
            pltpu.VMEM((1, 1), jnp.float32),
        ],
    )(adj, wh1, s1a, s1b, maxb, whsum, Wt, W2, a2)

    out = pl.pallas_call(
        functools.partial(_l2_kernel, n),
        grid=(npad // BI2, npad // BJ2),
        in_specs=[
            pl.BlockSpec((BI2, BJ2), lambda i, j: (i, j)),
            pl.BlockSpec((npad, h2), lambda i, j: (0, 0)),
            pl.BlockSpec((BI2, h2), lambda i, j: (i, 0)),
            pl.BlockSpec((npad, c), lambda i, j: (0, 0)),
            pl.BlockSpec((BI2, 1), lambda i, j: (i, 0)),
            pl.BlockSpec((1, BJ2), lambda i, j: (0, j)),
            pl.BlockSpec((BI2, 1), lambda i, j: (i, 0)),
            pl.BlockSpec((1, 1), lambda i, j: (0, 0)),
            pl.BlockSpec((1, 1), lambda i, j: (0, 0)),
            pl.BlockSpec((1, c), lambda i, j: (0, 0)),
            pl.BlockSpec((1, BJ2), lambda i, j: (0, j)),
        ],
        out_specs=pl.BlockSpec((BI2, c), lambda i, j: (i, 0)),
        out_shape=jax.ShapeDtypeStruct((npad, c), jnp.float32),
        scratch_shapes=[
            pltpu.VMEM((BI2, c), jnp.float32),
            pltpu.VMEM((BI2, 1), jnp.float32),
            pltpu.VMEM((BI2, 1), jnp.float32),
        ],
    )(adj, res, xwt, wh2, s2a, s2b, normx, gnorm, maxs2b, wh2sum, zcol)

    return out[:n]


# speedup vs baseline: 3.7354x; 1.0474x over previous
import functools

import jax
import jax.numpy as jnp
from jax.experimental import pallas as pl
from jax.experimental.pallas import tpu as pltpu

ALPHA_SLOPE = 0.2
NEG_INF = float("-inf")
LOG2E = 1.4426950408889634

BI1, BJ1 = 256, 1024
BI2, BJ2 = 256, 1024
BI0 = 512


def _leaky(x):
    return jnp.maximum(x, ALPHA_SLOPE * x)


def _proj_kernel(n_rows, x_ref, w1_ref, a1_ref,
                 wh_ref, s1a_ref, s1b_ref, maxb_ref, whsum_ref,
                 whsum_s, maxb_s):
    i = pl.program_id(0)
    ni = pl.num_programs(0)
    f = w1_ref.shape[1]
    bi = x_ref.shape[0]
    rows = i * bi + jax.lax.broadcasted_iota(jnp.int32, (bi, 1), 0)
    wh = jnp.dot(x_ref[...], w1_ref[...], preferred_element_type=jnp.float32)
    wh = jnp.where(rows < n_rows, wh, 0.0)
    pad = wh_ref.shape[1] - f
    lane = jax.lax.broadcasted_iota(jnp.int32, (bi, pad), 1)
    ones_col = jnp.where(lane == 0, jnp.float32(1.0), 0.0)
    wh_ref[...] = jnp.concatenate(
        [wh, ones_col], axis=1).astype(jnp.bfloat16)
    s1a_ref[...] = LOG2E * jnp.dot(wh, a1_ref[0:f, :],
                                   preferred_element_type=jnp.float32)
    s1b = LOG2E * jax.lax.dot_general(a1_ref[f:, :], wh, (((0,), (1,)), ((), ())),
                                      preferred_element_type=jnp.float32)
    cols = i * bi + jax.lax.broadcasted_iota(jnp.int32, (1, bi), 1)
    s1b = jnp.where(cols < n_rows, s1b, NEG_INF)
    s1b_ref[...] = s1b

    @pl.when(i == 0)
    def _():
        whsum_s[...] = jnp.zeros_like(whsum_s)
        maxb_s[...] = jnp.full_like(maxb_s, NEG_INF)

    whsum_s[...] += jnp.sum(wh, axis=0, keepdims=True)
    maxb_s[...] = jnp.maximum(maxb_s[...], jnp.max(s1b, axis=1, keepdims=True))

    @pl.when(i == ni - 1)
    def _():
        whsum_ref[...] = whsum_s[...]
        maxb_ref[...] = maxb_s[...]


def _l1_kernel(n_rows, adj_ref, wh_ref, s1a_ref, s1b_ref, maxb_ref, whsum_ref,
               wt_ref, w2_ref, a2_ref,
               res_ref, xwt_ref, wh2_ref, s2a_ref, s2b_ref, normx_ref,
               gnorm_ref, maxs2b_ref, wh2sum_ref,
               acc_ref, wh2s_s, gnorm_s, maxs2b_s):
    i = pl.program_id(0)
    j = pl.program_id(1)
    ni = pl.num_programs(0)
    nj = pl.num_programs(1)
    bi, bj = adj_ref.shape
    h2 = wt_ref.shape[0]
    c = w2_ref.shape[1]

    @pl.when(j == 0)
    def _():
        acc_ref[...] = jnp.zeros_like(acc_ref)

    m0 = _leaky(s1a_ref[...] + maxb_ref[...])
    e = _leaky(s1a_ref[...] + s1b_ref[...])
    p = jnp.exp2(jnp.where(adj_ref[...] > 0, e, NEG_INF) - m0)
    whj = wh_ref[pl.ds(j * bj, bj), :]
    acc_ref[...] += jnp.dot(p.astype(jnp.bfloat16), whj,
                            preferred_element_type=jnp.float32)

    @pl.when(j == nj - 1)
    def _():
        rows = i * bi + jax.lax.broadcasted_iota(jnp.int32, (bi, 1), 0)
        l = acc_ref[:, h2:h2 + 1]
        h = jnp.where(l > 0, acc_ref[:, 0:h2] / l, whsum_ref[...] / n_rows)
        h = jnp.where(h > 0, h, jnp.exp(h) - 1.0)
        h = jnp.where(rows < n_rows, h, 0.0)
        hb = h.astype(jnp.bfloat16)
        res_ref[...] = hb
        rn2 = hb.astype(jnp.float32)
        gn = jnp.sqrt(jnp.sum(rn2 * rn2, axis=1, keepdims=True))
        xwtb = (LOG2E * jnp.dot(h, wt_ref[...],
                                preferred_element_type=jnp.float32)
                ).astype(jnp.bfloat16)
        xwt_ref[...] = xwtb
        xw2 = xwtb.astype(jnp.float32)
        normx_ref[...] = jnp.sqrt(jnp.sum(xw2 * xw2, axis=1, keepdims=True))
        wh2 = jnp.dot(h, w2_ref[...], preferred_element_type=jnp.float32)
        wh2_ref[...] = wh2.astype(jnp.bfloat16)
        s2a_ref[...] = LOG2E * jnp.dot(wh2, a2_ref[0:c, :],
                                       preferred_element_type=jnp.float32)
        s2b = LOG2E * jax.lax.dot_general(a2_ref[c:, :], wh2,
                                          (((0,), (1,)), ((), ())),
                                          preferred_element_type=jnp.float32)
        cols = i * bi + jax.lax.broadcasted_iota(jnp.int32, (1, bi), 1)
        s2b = jnp.where(cols < n_rows, s2b, NEG_INF)
        s2b_ref[...] = s2b

        @pl.when(i == 0)
        def _():
            wh2s_s[...] = jnp.zeros_like(wh2s_s)
            gnorm_s[...] = jnp.zeros_like(gnorm_s)
            maxs2b_s[...] = jnp.full_like(maxs2b_s, NEG_INF)

        wh2s_s[...] += jnp.sum(wh2, axis=0, keepdims=True)
        gnorm_s[...] = jnp.maximum(
            gnorm_s[...], jnp.max(gn, axis=0, keepdims=True))
        maxs2b_s[...] = jnp.maximum(
            maxs2b_s[...], jnp.max(s2b, axis=1, keepdims=True))

        @pl.when(i == ni - 1)
        def _():
            wh2sum_ref[...] = wh2s_s[...]
            gnorm_ref[...] = gnorm_s[...]
            maxs2b_ref[...] = maxs2b_s[...]


def _l2_kernel(n_rows, adj_ref, res_ref, xwt_ref, wh2_ref, s2a_ref, s2b_ref,
               normx_ref, gnorm_ref, maxs2b_ref, wh2sum_ref, zcol_ref,
               out_ref, acc_ref, lt_ref, le_ref):
    j = pl.program_id(1)
    nj = pl.num_programs(1)
    bj = adj_ref.shape[1]

    @pl.when(j == 0)
    def _():
        acc_ref[...] = jnp.zeros_like(acc_ref)
        lt_ref[...] = jnp.zeros_like(lt_ref)
        le_ref[...] = jnp.zeros_like(le_ref)

    mt = _leaky(normx_ref[...] * gnorm_ref[...])
    me = _leaky(s2a_ref[...] + maxs2b_ref[...])
    mask = adj_ref[...] > 0
    xj = res_ref[pl.ds(j * bj, bj), :]
    t = jax.lax.dot_general(xwt_ref[...], xj, (((1,), (1,)), ((), ())),
                            preferred_element_type=jnp.float32)
    pt = jnp.exp2(jnp.where(mask, _leaky(t) + zcol_ref[...], NEG_INF) - mt)
    e2 = _leaky(s2a_ref[...] + s2b_ref[...])
    pe = jnp.exp2(jnp.where(mask, e2, NEG_INF) - me)
    lt_ref[...] += jnp.sum(pt, axis=1, keepdims=True)
    le_ref[...] += jnp.sum(pe, axis=1, keepdims=True)
    wh2j = wh2_ref[pl.ds(j * bj, bj), :]
    acc_ref[...] += jnp.dot((pt * pe).astype(jnp.bfloat16), wh2j,
                            preferred_element_type=jnp.float32)

    @pl.when(j == nj - 1)
    def _():
        denom = lt_ref[...] * le_ref[...]
        uni = wh2sum_ref[...] / jnp.float32(n_rows * n_rows)
        h = jnp.where(denom > 0, acc_ref[...] / denom, uni)
        mx = jnp.max(h, axis=1, keepdims=True)
        lse = mx + jnp.log(jnp.sum(jnp.exp(h - mx), axis=1, keepdims=True))
        out_ref[...] = h - lse


@jax.jit
def kernel(inputs, adj, W1, a1, Wt, W2, a2):
    n, feat = inputs.shape
    h2 = W1.shape[1]
    h2p = h2 + 8
    c = W2.shape[1]
    npad = pl.cdiv(n, BJ1) * BJ1

    ids = jax.lax.iota(jnp.int32, npad)[None, :]
    zcol = jnp.where(ids < n, jnp.float32(0.0), NEG_INF)

    wh1, s1a, s1b, maxb, whsum = pl.pallas_call(
        functools.partial(_proj_kernel, n),
        grid=(npad // BI0,),
        in_specs=[
            pl.BlockSpec((BI0, feat), lambda i: (i, 0)),
            pl.BlockSpec((feat, h2), lambda i: (0, 0)),
            pl.BlockSpec((2 * h2, 1), lambda i: (0, 0)),
        ],
        out_specs=[
            pl.BlockSpec((BI0, h2p), lambda i: (i, 0)),
            pl.BlockSpec((BI0, 1), lambda i: (i, 0)),
            pl.BlockSpec((1, BI0), lambda i: (0, i)),
            pl.BlockSpec((1, 1), lambda i: (0, 0)),
            pl.BlockSpec((1, h2), lambda i: (0, 0)),
        ],
        out_shape=[
            jax.ShapeDtypeStruct((npad, h2p), jnp.bfloat16),
            jax.ShapeDtypeStruct((npad, 1), jnp.float32),
            jax.ShapeDtypeStruct((1, npad), jnp.float32),
            jax.ShapeDtypeStruct((1, 1), jnp.float32),
            jax.ShapeDtypeStruct((1, h2), jnp.float32),
        ],
        scratch_shapes=[
            pltpu.VMEM((1, h2), jnp.float32),
            pltpu.VMEM((1, 1), jnp.float32),
        ],
    )(inputs, W1, a1)

    (res, xwt, wh2, s2a, s2b, normx, gnorm, maxs2b, wh2sum) = pl.pallas_call(
        functools.partial(_l1_kernel, n),
        grid=(npad // BI1, npad // BJ1),
        in_specs=[
            pl.BlockSpec((BI1, BJ1), lambda i, j: (i, j)),
            pl.BlockSpec((npad, h2p), lambda i, j: (0, 0)),
            pl.BlockSpec((BI1, 1), lambda i, j: (i, 0)),
            pl.BlockSpec((1, BJ1), lambda i, j: (0, j)),
            pl.BlockSpec((1, 1), lambda i, j: (0, 0)),
            pl.BlockSpec((1, h2), lambda i, j: (0, 0)),
            pl.BlockSpec((h2, h2), lambda i, j: (0, 0)),
            pl.BlockSpec((h2, c), lambda i, j: (0, 0)),
            pl.BlockSpec((2 * c, 1), lambda i, j: (0, 0)),
        ],
        out_specs=[
            pl.BlockSpec((BI1, h2), lambda i, j: (i, 0)),
            pl.BlockSpec((BI1, h2), lambda i, j: (i, 0)),
            pl.BlockSpec((BI1, c), lambda i, j: (i, 0)),
            pl.BlockSpec((BI1, 1), lambda i, j: (i, 0)),
            pl.BlockSpec((1, BI1), lambda i, j: (0, i)),
            pl.BlockSpec((BI1, 1), lambda i, j: (i, 0)),
            pl.BlockSpec((1, 1), lambda i, j: (0, 0)),
            pl.BlockSpec((1, 1), lambda i, j: (0, 0)),
            pl.BlockSpec((1, c), lambda i, j: (0, 0)),
        ],
        out_shape=[
            jax.ShapeDtypeStruct((npad, h2), jnp.bfloat16),
            jax.ShapeDtypeStruct((npad, h2), jnp.bfloat16),
            jax.ShapeDtypeStruct((npad, c), jnp.bfloat16),
            jax.ShapeDtypeStruct((npad, 1), jnp.float32),
            jax.ShapeDtypeStruct((1, npad), jnp.float32),
            jax.ShapeDtypeStruct((npad, 1), jnp.float32),
            jax.ShapeDtypeStruct((1, 1), jnp.float32),
            jax.ShapeDtypeStruct((1, 1), jnp.float32),
            jax.ShapeDtypeStruct((1, c), jnp.float32),
        ],
        scratch_shapes=[
            pltpu.VMEM((BI1, h2p), jnp.float32),
            pltpu.VMEM((1, c), jnp.float32),
            pltpu.VMEM((1, 1), jnp.float32),
            pltpu.VMEM((1, 1), jnp.float32),
        ],
    )(adj, wh1, s1a, s1b, maxb, whsum, Wt, W2, a2)

    out = pl.pallas_call(
        functools.partial(_l2_kernel, n),
        grid=(npad // BI2, npad // BJ2),
        in_specs=[
            pl.BlockSpec((BI2, BJ2), lambda i, j: (i, j)),
            pl.BlockSpec((npad, h2), lambda i, j: (0, 0)),
            pl.BlockSpec((BI2, h2), lambda i, j: (i, 0)),
            pl.BlockSpec((npad, c), lambda i, j: (0, 0)),
            pl.BlockSpec((BI2, 1), lambda i, j: (i, 0)),
            pl.BlockSpec((1, BJ2), lambda i, j: (0, j)),
            pl.BlockSpec((BI2, 1), lambda i, j: (i, 0)),
            pl.BlockSpec((1, 1), lambda i, j: (0, 0)),
            pl.BlockSpec((1, 1), lambda i, j: (0, 0)),
            pl.BlockSpec((1, c), lambda i, j: (0, 0)),
            pl.BlockSpec((1, BJ2), lambda i, j: (0, j)),
        ],
        out_specs=pl.BlockSpec((BI2, c), lambda i, j: (i, 0)),
        out_shape=jax.ShapeDtypeStruct((npad, c), jnp.float32),
        scratch_shapes=[
            pltpu.VMEM((BI2, c), jnp.float32),
            pltpu.VMEM((BI2, 1), jnp.float32),
            pltpu.VMEM((BI2, 1), jnp.float32),
        ],
    )(adj, res, xwt, wh2, s2a, s2b, normx, gnorm, maxs2b, wh2sum, zcol)

    return out[:n]


# speedup vs baseline: 5.0643x; 1.3558x over previous
import functools

import jax
import jax.numpy as jnp
from jax.experimental import pallas as pl
from jax.experimental.pallas import tpu as pltpu

ALPHA_SLOPE = 0.2
NEG_INF = float("-inf")
LOG2E = 1.4426950408889634

BI1, BJ1 = 512, 1024
BI2, BJ2 = 512, 1024
BI0 = 512


def _leaky(x):
    return jnp.maximum(x, ALPHA_SLOPE * x)


def _proj_kernel(n_rows, x_ref, w1_ref, a1_ref,
                 wh_ref, s1a_ref, s1b_ref, maxb_ref, whsum_ref,
                 whsum_s, maxb_s):
    i = pl.program_id(0)
    ni = pl.num_programs(0)
    f = w1_ref.shape[1]
    bi = x_ref.shape[0]
    rows = i * bi + jax.lax.broadcasted_iota(jnp.int32, (bi, 1), 0)
    wh = jnp.dot(x_ref[...], w1_ref[...], preferred_element_type=jnp.float32)
    wh = jnp.where(rows < n_rows, wh, 0.0)
    pad = wh_ref.shape[1] - f
    lane = jax.lax.broadcasted_iota(jnp.int32, (bi, pad), 1)
    ones_col = jnp.where(lane == 0, jnp.float32(1.0), 0.0)
    wh_ref[...] = jnp.concatenate(
        [wh, ones_col], axis=1).astype(jnp.bfloat16)
    s1a_ref[...] = LOG2E * jnp.dot(wh, a1_ref[0:f, :],
                                   preferred_element_type=jnp.float32)
    s1b = LOG2E * jax.lax.dot_general(a1_ref[f:, :], wh, (((0,), (1,)), ((), ())),
                                      preferred_element_type=jnp.float32)
    cols = i * bi + jax.lax.broadcasted_iota(jnp.int32, (1, bi), 1)
    s1b = jnp.where(cols < n_rows, s1b, NEG_INF)
    s1b_ref[...] = s1b

    @pl.when(i == 0)
    def _():
        whsum_s[...] = jnp.zeros_like(whsum_s)
        maxb_s[...] = jnp.full_like(maxb_s, NEG_INF)

    whsum_s[...] += jnp.sum(wh, axis=0, keepdims=True)
    maxb_s[...] = jnp.maximum(maxb_s[...], jnp.max(s1b, axis=1, keepdims=True))

    @pl.when(i == ni - 1)
    def _():
        whsum_ref[...] = whsum_s[...]
        maxb_ref[...] = maxb_s[...]


def _l1_kernel(n_rows, adj_ref, wh_ref, s1a_ref, s1b_ref, maxb_ref, whsum_ref,
               wt_ref, w2_ref, a2_ref,
               res_ref, xwt_ref, wh2_ref, s2a_ref, s2b_ref, normx_ref,
               gnorm_ref, maxs2b_ref, wh2sum_ref,
               acc_ref, wh2s_s, gnorm_s, maxs2b_s):
    i = pl.program_id(0)
    j = pl.program_id(1)
    ni = pl.num_programs(0)
    nj = pl.num_programs(1)
    bi, bj = adj_ref.shape
    h2 = wt_ref.shape[0]
    c = w2_ref.shape[1]

    @pl.when(j == 0)
    def _():
        acc_ref[...] = jnp.zeros_like(acc_ref)

    m0 = _leaky(s1a_ref[...] + maxb_ref[...])
    e = _leaky(s1a_ref[...] + s1b_ref[...])
    p = jnp.exp2(jnp.where(adj_ref[...] > 0, e, NEG_INF) - m0)
    whj = wh_ref[pl.ds(j * bj, bj), :]
    acc_ref[...] += jnp.dot(p.astype(jnp.bfloat16), whj,
                            preferred_element_type=jnp.float32)

    @pl.when(j == nj - 1)
    def _():
        rows = i * bi + jax.lax.broadcasted_iota(jnp.int32, (bi, 1), 0)
        l = acc_ref[:, h2:h2 + 1]
        h = jnp.where(l > 0, acc_ref[:, 0:h2] / l, whsum_ref[...] / n_rows)
        h = jnp.where(h > 0, h, jnp.exp(h) - 1.0)
        h = jnp.where(rows < n_rows, h, 0.0)
        hb = h.astype(jnp.bfloat16)
        res_ref[...] = hb
        rn2 = hb.astype(jnp.float32)
        gn = jnp.sqrt(jnp.sum(rn2 * rn2, axis=1, keepdims=True))
        xwtb = (LOG2E * jnp.dot(h, wt_ref[...],
                                preferred_element_type=jnp.float32)
                ).astype(jnp.bfloat16)
        xwt_ref[...] = xwtb
        xw2 = xwtb.astype(jnp.float32)
        normx_ref[...] = jnp.sqrt(jnp.sum(xw2 * xw2, axis=1, keepdims=True))
        wh2 = jnp.dot(h, w2_ref[...], preferred_element_type=jnp.float32)
        wh2_ref[...] = wh2.astype(jnp.bfloat16)
        s2a_ref[...] = LOG2E * jnp.dot(wh2, a2_ref[0:c, :],
                                       preferred_element_type=jnp.float32)
        s2b = LOG2E * jax.lax.dot_general(a2_ref[c:, :], wh2,
                                          (((0,), (1,)), ((), ())),
                                          preferred_element_type=jnp.float32)
        cols = i * bi + jax.lax.broadcasted_iota(jnp.int32, (1, bi), 1)
        s2b = jnp.where(cols < n_rows, s2b, NEG_INF)
        s2b_ref[...] = s2b

        @pl.when(i == 0)
        def _():
            wh2s_s[...] = jnp.zeros_like(wh2s_s)
            gnorm_s[...] = jnp.zeros_like(gnorm_s)
            maxs2b_s[...] = jnp.full_like(maxs2b_s, NEG_INF)

        wh2s_s[...] += jnp.sum(wh2, axis=0, keepdims=True)
        gnorm_s[...] = jnp.maximum(
            gnorm_s[...], jnp.max(gn, axis=0, keepdims=True))
        maxs2b_s[...] = jnp.maximum(
            maxs2b_s[...], jnp.max(s2b, axis=1, keepdims=True))

        @pl.when(i == ni - 1)
        def _():
            wh2sum_ref[...] = wh2s_s[...]
            gnorm_ref[...] = gnorm_s[...]
            maxs2b_ref[...] = maxs2b_s[...]


def _l2_kernel(n_rows, adj_ref, res_ref, xwt_ref, wh2_ref, s2a_ref, s2b_ref,
               normx_ref, gnorm_ref, maxs2b_ref, wh2sum_ref, zcol_ref,
               out_ref, acc_ref, lt_ref, le_ref):
    j = pl.program_id(1)
    nj = pl.num_programs(1)
    bj = adj_ref.shape[1]

    @pl.when(j == 0)
    def _():
        acc_ref[...] = jnp.zeros_like(acc_ref)
        lt_ref[...] = jnp.zeros_like(lt_ref)
        le_ref[...] = jnp.zeros_like(le_ref)

    mt = _leaky(normx_ref[...] * gnorm_ref[...])
    me = _leaky(s2a_ref[...] + maxs2b_ref[...])
    mask = adj_ref[...] > 0
    xj = res_ref[pl.ds(j * bj, bj), :]
    t = jax.lax.dot_general(xwt_ref[...], xj, (((1,), (1,)), ((), ())),
                            preferred_element_type=jnp.float32)
    pt = jnp.exp2(jnp.where(mask, _leaky(t) + zcol_ref[...], NEG_INF) - mt)
    e2 = _leaky(s2a_ref[...] + s2b_ref[...])
    pe = jnp.exp2(jnp.where(mask, e2, NEG_INF) - me)
    lt_ref[...] += jnp.sum(pt, axis=1, keepdims=True)
    le_ref[...] += jnp.sum(pe, axis=1, keepdims=True)
    wh2j = wh2_ref[pl.ds(j * bj, bj), :]
    acc_ref[...] += jnp.dot((pt * pe).astype(jnp.bfloat16), wh2j,
                            preferred_element_type=jnp.float32)

    @pl.when(j == nj - 1)
    def _():
        denom = lt_ref[...] * le_ref[...]
        uni = wh2sum_ref[...] / jnp.float32(n_rows * n_rows)
        h = jnp.where(denom > 0, acc_ref[...] / denom, uni)
        mx = jnp.max(h, axis=1, keepdims=True)
        lse = mx + jnp.log(jnp.sum(jnp.exp(h - mx), axis=1, keepdims=True))
        out_ref[...] = h - lse


@jax.jit
def kernel(inputs, adj, W1, a1, Wt, W2, a2):
    n, feat = inputs.shape
    h2 = W1.shape[1]
    h2p = h2 + 8
    c = W2.shape[1]
    npad = pl.cdiv(n, BJ1) * BJ1

    ids = jax.lax.iota(jnp.int32, npad)[None, :]
    zcol = jnp.where(ids < n, jnp.float32(0.0), NEG_INF)

    wh1, s1a, s1b, maxb, whsum = pl.pallas_call(
        functools.partial(_proj_kernel, n),
        grid=(npad // BI0,),
        in_specs=[
            pl.BlockSpec((BI0, feat), lambda i: (i, 0)),
            pl.BlockSpec((feat, h2), lambda i: (0, 0)),
            pl.BlockSpec((2 * h2, 1), lambda i: (0, 0)),
        ],
        out_specs=[
            pl.BlockSpec((BI0, h2p), lambda i: (i, 0)),
            pl.BlockSpec((BI0, 1), lambda i: (i, 0)),
            pl.BlockSpec((1, BI0), lambda i: (0, i)),
            pl.BlockSpec((1, 1), lambda i: (0, 0)),
            pl.BlockSpec((1, h2), lambda i: (0, 0)),
        ],
        out_shape=[
            jax.ShapeDtypeStruct((npad, h2p), jnp.bfloat16),
            jax.ShapeDtypeStruct((npad, 1), jnp.float32),
            jax.ShapeDtypeStruct((1, npad), jnp.float32),
            jax.ShapeDtypeStruct((1, 1), jnp.float32),
            jax.ShapeDtypeStruct((1, h2), jnp.float32),
        ],
        scratch_shapes=[
            pltpu.VMEM((1, h2), jnp.float32),
            pltpu.VMEM((1, 1), jnp.float32),
        ],
    )(inputs, W1, a1)

    (res, xwt, wh2, s2a, s2b, normx, gnorm, maxs2b, wh2sum) = pl.pallas_call(
        functools.partial(_l1_kernel, n),
        grid=(npad // BI1, npad // BJ1),
        in_specs=[
            pl.BlockSpec((BI1, BJ1), lambda i, j: (i, j)),
            pl.BlockSpec((npad, h2p), lambda i, j: (0, 0)),
            pl.BlockSpec((BI1, 1), lambda i, j: (i, 0)),
            pl.BlockSpec((1, BJ1), lambda i, j: (0, j)),
            pl.BlockSpec((1, 1), lambda i, j: (0, 0)),
            pl.BlockSpec((1, h2), lambda i, j: (0, 0)),
            pl.BlockSpec((h2, h2), lambda i, j: (0, 0)),
            pl.BlockSpec((h2, c), lambda i, j: (0, 0)),
            pl.BlockSpec((2 * c, 1), lambda i, j: (0, 0)),
        ],
        out_specs=[
            pl.BlockSpec((BI1, h2), lambda i, j: (i, 0)),
            pl.BlockSpec((BI1, h2), lambda i, j: (i, 0)),
            pl.BlockSpec((BI1, c), lambda i, j: (i, 0)),
            pl.BlockSpec((BI1, 1), lambda i, j: (i, 0)),
            pl.BlockSpec((1, BI1), lambda i, j: (0, i)),
            pl.BlockSpec((BI1, 1), lambda i, j: (i, 0)),
            pl.BlockSpec((1, 1), lambda i, j: (0, 0)),
            pl.BlockSpec((1, 1), lambda i, j: (0, 0)),
            pl.BlockSpec((1, c), lambda i, j: (0, 0)),
        ],
        out_shape=[
            jax.ShapeDtypeStruct((npad, h2), jnp.bfloat16),
            jax.ShapeDtypeStruct((npad, h2), jnp.bfloat16),
            jax.ShapeDtypeStruct((npad, c), jnp.bfloat16),
            jax.ShapeDtypeStruct((npad, 1), jnp.float32),
            jax.ShapeDtypeStruct((1, npad), jnp.float32),
            jax.ShapeDtypeStruct((npad, 1), jnp.float32),
            jax.ShapeDtypeStruct((1, 1), jnp.float32),
            jax.ShapeDtypeStruct((1, 1), jnp.float32),
            jax.ShapeDtypeStruct((1, c), jnp.float32),
        ],
        scratch_shapes=[
            pltpu.VMEM((BI1, h2p), jnp.float32),
            pltpu.VMEM((1, c), jnp.float32),
            pltpu.VMEM((1, 1), jnp.float32),
            pltpu.VMEM((1, 1), jnp.float32),
        ],
    )(adj, wh1, s1a, s1b, maxb, whsum, Wt, W2, a2)

    out = pl.pallas_call(
        functools.partial(_l2_kernel, n),
        grid=(npad // BI2, npad // BJ2),
        in_specs=[
            pl.BlockSpec((BI2, BJ2), lambda i, j: (i, j)),
            pl.BlockSpec((npad, h2), lambda i, j: (0, 0)),
            pl.BlockSpec((BI2, h2), lambda i, j: (i, 0)),
            pl.BlockSpec((npad, c), lambda i, j: (0, 0)),
            pl.BlockSpec((BI2, 1), lambda i, j: (i, 0)),
            pl.BlockSpec((1, BJ2), lambda i, j: (0, j)),
            pl.BlockSpec((BI2, 1), lambda i, j: (i, 0)),
            pl.BlockSpec((1, 1), lambda i, j: (0, 0)),
            pl.BlockSpec((1, 1), lambda i, j: (0, 0)),
            pl.BlockSpec((1, c), lambda i, j: (0, 0)),
            pl.BlockSpec((1, BJ2), lambda i, j: (0, j)),
        ],
        out_specs=pl.BlockSpec((BI2, c), lambda i, j: (i, 0)),
        out_shape=jax.ShapeDtypeStruct((npad, c), jnp.float32),
        scratch_shapes=[
            pltpu.VMEM((BI2, c), jnp.float32),
            pltpu.VMEM((BI2, 1), jnp.float32),
            pltpu.VMEM((BI2, 1), jnp.float32),
        ],
    )(adj, res, xwt, wh2, s2a, s2b, normx, gnorm, maxs2b, wh2sum, zcol)

    return out[:n]
